# baseline (speedup 1.0000x reference)
import functools

import jax
import jax.numpy as jnp
from jax import lax
from jax.experimental import pallas as pl
from jax.experimental.pallas import tpu as pltpu
from jax.experimental.pallas import tpu_sc as plsc

DIM = 64

NC = 2
NS = 16
NW = NC * NS

NBUF = 4


def _split_chunks(l: int):
    chunks = []
    off = 0
    while off < l:
        n = min(128, l - off)
        chunks.append((off, n))
        off += n
    assert all(o % 8 == 0 for o, _ in chunks[:-1])
    return chunks


def _make_lookup(bsz: int, l: int):
    rows_per_w = bsz // NW
    assert rows_per_w % NBUF == 0 and rows_per_w >= 2 * NBUF
    chunks = _split_chunks(l)

    mesh = plsc.VectorSubcoreMesh(core_axis_name="c", subcore_axis_name="s")

    @functools.partial(
        pl.kernel,
        out_type=jax.ShapeDtypeStruct((bsz, l, DIM), jnp.float32),
        mesh=mesh,
        scratch_types=[
            pltpu.VMEM((rows_per_w, l), jnp.int32),
            [pltpu.VMEM((l, DIM), jnp.float32) for _ in range(NBUF)],
            [pltpu.SemaphoreType.DMA for _ in range(NBUF)],
            [pltpu.SemaphoreType.DMA for _ in range(NBUF)],
        ],
        compiler_params=pltpu.CompilerParams(use_tc_tiling_on_sc=False),
    )
    def lookup(table_hbm, idx_hbm, out_hbm, idx_v, rows, gsem, wsem):
        wid = lax.axis_index("s") * NC + lax.axis_index("c")
        base = wid * rows_per_w
        pltpu.sync_copy(idx_hbm.at[pl.ds(base, rows_per_w)], idx_v)

        def gather_start(i, b):
            for off, n in chunks:
                pltpu.async_copy(
                    table_hbm.at[idx_v.at[i, pl.ds(off, n)]],
                    rows[b].at[pl.ds(off, n)],
                    gsem[b],
                )

        def gather_wait(b):
            pltpu.make_async_copy(
                table_hbm.at[pl.ds(0, l)], rows[b], gsem[b]
            ).wait()

        def writeback_start(i, b):
            return pltpu.async_copy(rows[b], out_hbm.at[base + i], wsem[b])

        for b in range(NBUF):
            gather_start(b, b)

        @pl.loop(0, rows_per_w - NBUF, step=NBUF)
        def body(i0):
            for b in range(NBUF):
                i = i0 + b
                gather_wait(b)
                writeback_start(i, b).wait()
                gather_start(i + NBUF, b)

        for b in range(NBUF):
            i = rows_per_w - NBUF + b
            gather_wait(b)
            writeback_start(i, b)
        for b in range(NBUF):
            i = rows_per_w - NBUF + b
            pltpu.make_async_copy(rows[b], out_hbm.at[base + i], wsem[b]).wait()

    return lookup


def kernel(input, weight):
    bsz, l = input.shape
    return _make_lookup(bsz, l)(weight, input.astype(jnp.int32))
